# baseline (speedup 1.0000x reference)
import functools
import jax
import jax.numpy as jnp
from jax import lax
from jax.experimental import pallas as pl
from jax.experimental.pallas import tpu as pltpu
from jax.experimental.pallas import tpu_sc as plsc

NC = 2
NS = 16
NW = NC * NS

VOCAB_ROWS = 1_000_000
DIM = 64
BATCH = 4096 * 200
B_PER_W = BATCH // NW
CHUNK = 128
N_CHUNKS = B_PER_W // CHUNK
NBUF = 2


def _emb_body(x_hbm, table_hbm, out_hbm, idx_v, rows_v, gsems, osems):
    wid = lax.axis_index("s") * NC + lax.axis_index("c")
    base = wid * B_PER_W

    pltpu.sync_copy(x_hbm.at[wid], idx_v)

    def start_gather(g, buf):
        pltpu.async_copy(
            table_hbm.at[idx_v.at[g]],
            rows_v.at[buf],
            gsems.at[buf],
        )

    def wait_gather(g, buf):
        pltpu.make_async_copy(
            table_hbm.at[idx_v.at[g]],
            rows_v.at[buf],
            gsems.at[buf],
        ).wait()

    def start_out(g, buf):
        pltpu.async_copy(
            rows_v.at[buf],
            out_hbm.at[pl.ds(base + g * CHUNK, CHUNK)],
            osems.at[buf],
        )

    def wait_out(g, buf):
        pltpu.make_async_copy(
            rows_v.at[buf],
            out_hbm.at[pl.ds(base + g * CHUNK, CHUNK)],
            osems.at[buf],
        ).wait()

    for b in range(NBUF):
        start_gather(b, b)

    def outer(i, carry):
        g0 = i * NBUF
        for b in range(NBUF):
            g = g0 + b
            wait_gather(g, b)
            start_out(g, b)
            wait_out(g, b)

            @pl.when(g + NBUF < N_CHUNKS)
            def _():
                start_gather(g + NBUF, b)

        return carry

    lax.fori_loop(0, N_CHUNKS // NBUF, outer, 0, unroll=False)


@jax.jit
def _emb(x_flat, table):
    run = pl.kernel(
        _emb_body,
        out_type=jax.ShapeDtypeStruct((BATCH, DIM), jnp.float32),
        mesh=plsc.VectorSubcoreMesh(core_axis_name="c", subcore_axis_name="s"),
        scratch_types=[
            pltpu.VMEM((N_CHUNKS, CHUNK), jnp.int32),
            pltpu.VMEM((NBUF, CHUNK, DIM), jnp.float32),
            pltpu.SemaphoreType.DMA((NBUF,)),
            pltpu.SemaphoreType.DMA((NBUF,)),
        ],
        compiler_params=pltpu.CompilerParams(use_tc_tiling_on_sc=False),
    )
    return run(x_flat, table)


def kernel(x, table):
    out = _emb(x.reshape(NW, N_CHUNKS, CHUNK), table)
    return out.reshape(x.shape[0], x.shape[1], DIM)


# speedup vs baseline: 1.0224x; 1.0224x over previous
import functools
import jax
import jax.numpy as jnp
from jax import lax
from jax.experimental import pallas as pl
from jax.experimental.pallas import tpu as pltpu
from jax.experimental.pallas import tpu_sc as plsc

NC = 2
NS = 16
NW = NC * NS

VOCAB_ROWS = 1_000_000
DIM = 64
BATCH = 4096 * 200
B_PER_W = BATCH // NW
CHUNK = 128
N_CHUNKS = B_PER_W // CHUNK
K = 4
SUPER = CHUNK * K
N_SUPER = B_PER_W // SUPER
NBUF = 2


def _emb_body(x_hbm, table_hbm, out_hbm, idx_v, rows_v, gsems, osems):
    wid = lax.axis_index("s") * NC + lax.axis_index("c")
    base = wid * B_PER_W

    pltpu.sync_copy(x_hbm.at[wid], idx_v)

    def fire_gathers(g, buf):
        for j in range(K):
            pltpu.async_copy(
                table_hbm.at[idx_v.at[g * K + j]],
                rows_v.at[buf, pl.ds(j * CHUNK, CHUNK)],
                gsems.at[buf],
            )

    def drain_gathers(g, buf):
        for j in range(K):
            pltpu.make_async_copy(
                table_hbm.at[idx_v.at[g * K + j]],
                rows_v.at[buf, pl.ds(j * CHUNK, CHUNK)],
                gsems.at[buf],
            ).wait()

    def start_out(g, buf):
        pltpu.async_copy(
            rows_v.at[buf],
            out_hbm.at[pl.ds(base + g * SUPER, SUPER)],
            osems.at[buf],
        )

    def wait_out(g, buf):
        pltpu.make_async_copy(
            rows_v.at[buf],
            out_hbm.at[pl.ds(base + g * SUPER, SUPER)],
            osems.at[buf],
        ).wait()

    for b in range(NBUF):
        fire_gathers(b, b)

    def outer(i, carry):
        g0 = i * NBUF
        for b in range(NBUF):
            g = g0 + b
            drain_gathers(g, b)
            start_out(g, b)
            wait_out(g, b)

            @pl.when(g + NBUF < N_SUPER)
            def _():
                fire_gathers(g + NBUF, b)

        return carry

    lax.fori_loop(0, N_SUPER // NBUF, outer, 0, unroll=False)


@jax.jit
def _emb(x_flat, table):
    run = pl.kernel(
        _emb_body,
        out_type=jax.ShapeDtypeStruct((BATCH, DIM), jnp.float32),
        mesh=plsc.VectorSubcoreMesh(core_axis_name="c", subcore_axis_name="s"),
        scratch_types=[
            pltpu.VMEM((N_CHUNKS, CHUNK), jnp.int32),
            pltpu.VMEM((NBUF, SUPER, DIM), jnp.float32),
            pltpu.SemaphoreType.DMA((NBUF,)),
            pltpu.SemaphoreType.DMA((NBUF,)),
        ],
        compiler_params=pltpu.CompilerParams(use_tc_tiling_on_sc=False),
    )
    return run(x_flat, table)


def kernel(x, table):
    out = _emb(x.reshape(NW, N_CHUNKS, CHUNK), table)
    return out.reshape(x.shape[0], x.shape[1], DIM)
